# baseline (speedup 1.0000x reference)
import functools

import jax
import jax.numpy as jnp
from jax import lax
from jax.experimental import pallas as pl
from jax.experimental.pallas import tpu as pltpu
from jax.experimental.pallas import tpu_sc as plsc

_BIG = 2e30
_MASKED = 3e30
_PADV = 1e30
_IBIG = 2**31 - 1
_EPS = 1e-5


def _topk_body(n_valid, k_sel, cc, xrow_ref, xtt_ref, xxr_ref, xxc_ref,
               w1t_ref, w2t_ref, idx_ref, y1_ref, y2_ref):
    xrow = xrow_ref[...]
    dot_hi = functools.partial(
        jax.lax.dot_general,
        dimension_numbers=(((1,), (0,)), ((), ())),
        precision=jax.lax.Precision.HIGHEST,
        preferred_element_type=jnp.float32,
    )
    dot_lo = functools.partial(
        jax.lax.dot_general,
        dimension_numbers=(((1,), (0,)), ((), ())),
        preferred_element_type=jnp.float32,
    )
    y1_ref[...] = dot_hi(xrow, w1t_ref[...])
    y2_ref[...] = dot_hi(xrow, w2t_ref[...])

    rb = xrow.shape[0]
    np_ = xtt_ref.shape[1]
    half = np_ // 2
    xxr = xxr_ref[...]
    inner = dot_lo(xrow, xtt_ref[...])
    xxc = xxc_ref[...]
    colv = jax.lax.broadcasted_iota(jnp.int32, (1, np_), 1)
    d = (xxr + xxc) - 2.0 * inner
    d = jnp.where(colv >= n_valid, _PADV, d)

    a, b = d[:, :half], d[:, half:]
    ia = jax.lax.broadcasted_iota(jnp.int32, (rb, half), 1)
    ib = ia + half
    sel_a = a <= b
    pmin = jnp.where(sel_a, a, b)
    pmax = jnp.where(sel_a, b, a)
    pmin_i = jnp.where(sel_a, ia, ib)
    isum = ia + ib

    new_i = []
    for _ in range(k_sel):
        m = jnp.min(pmin, axis=1, keepdims=True)
        cand = jnp.where(pmin == m, pmin_i, _IBIG)
        am = jnp.min(cand, axis=1, keepdims=True)
        hit = cand == am
        pmin = jnp.where(hit, pmax, pmin)
        pmin_i = jnp.where(hit, isum - pmin_i, pmin_i)
        pmax = jnp.where(hit, _MASKED, pmax)
        new_i.append(am)
    idx_ref[...] = jnp.concatenate(new_i, axis=1)


def _build_topk(np_, c, o, rb, cc, k_sel, n_valid, interpret=False):
    body = functools.partial(_topk_body, n_valid, k_sel, cc)
    return pl.pallas_call(
        body,
        grid=(np_ // rb,),
        in_specs=[
            pl.BlockSpec((rb, c), lambda i: (i, 0)),
            pl.BlockSpec((c, np_), lambda i: (0, 0)),
            pl.BlockSpec((rb, 1), lambda i: (i, 0)),
            pl.BlockSpec((1, np_), lambda i: (0, 0)),
            pl.BlockSpec((c, o), lambda i: (0, 0)),
            pl.BlockSpec((c, o), lambda i: (0, 0)),
        ],
        out_specs=[
            pl.BlockSpec((rb, k_sel), lambda i: (i, 0)),
            pl.BlockSpec((rb, o), lambda i: (i, 0)),
            pl.BlockSpec((rb, o), lambda i: (i, 0)),
        ],
        out_shape=[
            jax.ShapeDtypeStruct((np_, k_sel), jnp.int32),
            jax.ShapeDtypeStruct((np_, o), jnp.float32),
            jax.ShapeDtypeStruct((np_, o), jnp.float32),
        ],
        interpret=interpret,
    )


def _build_sc_gather(np_, o, k_sel, n_valid):
    nw = 32
    rpc = 8
    gpr = o // 16
    rpw = np_ // nw
    nchunks = rpw // rpc
    mesh = plsc.VectorSubcoreMesh(core_axis_name="c", subcore_axis_name="s")

    @functools.partial(
        pl.kernel,
        mesh=mesh,
        out_type=[
            jax.ShapeDtypeStruct((np_, o), jnp.float32),
            jax.ShapeDtypeStruct((np_, o), jnp.float32),
            jax.ShapeDtypeStruct((2 * nw, o), jnp.float32),
        ],
        scratch_types=[
            pltpu.VMEM((rpc * k_sel,), jnp.int32),
            pltpu.VMEM((rpc * k_sel, o), jnp.float32),
            pltpu.VMEM((rpc, o), jnp.float32),
            pltpu.VMEM((rpc, o), jnp.float32),
            pltpu.VMEM((rpc, o), jnp.float32),
            pltpu.VMEM((o,), jnp.float32),
            pltpu.VMEM((o,), jnp.float32),
            pltpu.SemaphoreType.DMA,
        ],
    )
    def sc_gather(y1_hbm, y2_hbm, idx_hbm, mx_hbm, mn_hbm, part_hbm,
                  idx_v, rows_v, y2_v, mx_v, mn_v, s_v, s2_v, sem):
        wid = lax.axis_index("s") * 2 + lax.axis_index("c")
        row0 = wid * rpw
        for g in range(gpr):
            s_v[pl.ds(g * 16, 16)] = jnp.zeros((16,), jnp.float32)
            s2_v[pl.ds(g * 16, 16)] = jnp.zeros((16,), jnp.float32)

        def chunk_body(ch, _):
            n0 = row0 + ch * rpc
            pltpu.sync_copy(idx_hbm.at[pl.ds(n0 * k_sel, rpc * k_sel)],
                            idx_v)
            pltpu.async_copy(y1_hbm.at[idx_v], rows_v, sem).wait()
            pltpu.sync_copy(y2_hbm.at[pl.ds(n0, rpc)], y2_v)

            def row_body(r, valid):
                for g in range(gpr):
                    cs = pl.ds(g * 16, 16)
                    y2g = y2_v[r, cs]
                    z = rows_v[r * k_sel, cs] + y2g
                    mx, mn, s, s2 = z, z, z, z * z
                    for k in range(1, k_sel):
                        z = rows_v[r * k_sel + k, cs] + y2g
                        mx = jnp.maximum(mx, z)
                        mn = jnp.minimum(mn, z)
                        s = s + z
                        s2 = s2 + z * z
                    mx_v[r, cs] = mx
                    mn_v[r, cs] = mn

                    @pl.when(valid)
                    def _():
                        s_v[cs] = s_v[cs] + s
                        s2_v[cs] = s2_v[cs] + s2
                return valid

            lax.fori_loop(0, rpc, row_body, n0 < n_valid)
            pltpu.sync_copy(mx_v, mx_hbm.at[pl.ds(n0, rpc)])
            pltpu.sync_copy(mn_v, mn_hbm.at[pl.ds(n0, rpc)])
            return ()

        lax.fori_loop(0, nchunks, chunk_body, ())
        pltpu.sync_copy(s_v, part_hbm.at[wid])
        pltpu.sync_copy(s2_v, part_hbm.at[nw + wid])

    return sc_gather


def _fin_body(m_count, mx_ref, mn_ref, part_ref, gamma_ref, beta_ref, out_ref):
    part = part_ref[...]
    wk = part.shape[0] // 2
    s = jnp.sum(part[:wk, :], axis=0, keepdims=True)
    s2 = jnp.sum(part[wk:, :], axis=0, keepdims=True)
    mean = s / m_count
    var = s2 / m_count - mean * mean
    scale = gamma_ref[...] * jax.lax.rsqrt(var + _EPS)
    shift = beta_ref[...] - mean * scale
    pick = jnp.where(scale >= 0.0, mx_ref[...], mn_ref[...])
    o = jnp.maximum(pick * scale + shift, 0.0)
    out_ref[0, ...] = o.T


def _build_fin(np_, o, rb, wk, m_count, interpret=False):
    body = functools.partial(_fin_body, float(m_count))
    return pl.pallas_call(
        body,
        grid=(np_ // rb,),
        in_specs=[
            pl.BlockSpec((rb, o), lambda i: (i, 0)),
            pl.BlockSpec((rb, o), lambda i: (i, 0)),
            pl.BlockSpec((2 * wk, o), lambda i: (0, 0)),
            pl.BlockSpec((1, o), lambda i: (0, 0)),
            pl.BlockSpec((1, o), lambda i: (0, 0)),
        ],
        out_specs=pl.BlockSpec((1, o, rb), lambda i: (0, 0, i)),
        out_shape=jax.ShapeDtypeStruct((1, o, np_), jnp.float32),
        interpret=interpret,
    )


def _edge_conv(x, W, gamma, beta, rb, cc, rb_fin, k_sel, interpret=False,
               use_sc=True):
    b, c, n = x.shape
    o = W.shape[0]
    xt = jnp.transpose(x[0])
    np_ = -(-n // cc) * cc
    xtp = jnp.pad(xt, ((0, np_ - n), (0, 0)))
    xtt = jnp.transpose(xtp)
    xx = jnp.sum(xt * xt, axis=1)
    xxr = jnp.pad(xx, (0, np_ - n))[:, None]
    xxc = jnp.pad(xx, (0, np_ - n))[None, :]
    w1t = jnp.transpose(W[:, :c])
    w2t = jnp.transpose(W[:, c:] - W[:, :c])

    idx, y1, y2 = _build_topk(np_, c, o, rb, cc, k_sel, n,
                              interpret=interpret)(xtp, xtt, xxr, xxc,
                                                   w1t, w2t)

    if use_sc:
        idxf = idx.reshape(np_ * k_sel)
        mx, mn, partials = _build_sc_gather(np_, o, k_sel, n)(y1, y2, idxf)
        wk = 32
    else:
        idxv = idx[:n]
        z = y1[idxv] + y2[:n, None, :]
        s = jnp.sum(z, axis=(0, 1))
        s2 = jnp.sum(z * z, axis=(0, 1))
        partials = jnp.stack([s, s2], axis=0)
        mx = jnp.pad(jnp.max(z, axis=1), ((0, np_ - n), (0, 0)))
        mn = jnp.pad(jnp.min(z, axis=1), ((0, np_ - n), (0, 0)))
        wk = 1

    out = _build_fin(np_, o, rb_fin, wk, n * k_sel, interpret=interpret)(
        mx, mn, partials, gamma[None, :], beta[None, :])
    return out[:, :, :n]


def kernel(x, W, gamma, beta):
    return _edge_conv(x, W, gamma, beta, rb=256, cc=2048, rb_fin=512,
                      k_sel=16)


# speedup vs baseline: 1.0880x; 1.0880x over previous
import functools

import jax
import jax.numpy as jnp
from jax import lax
from jax.experimental import pallas as pl
from jax.experimental.pallas import tpu as pltpu
from jax.experimental.pallas import tpu_sc as plsc

_BIG = 2e30
_MASKED = 3e30
_PADV = 1e30
_IBIG = 2**31 - 1
_EPS = 1e-5


def _topk_body(n_valid, k_sel, cc, xrow_ref, xtt_ref, xxr_ref, xxc_ref,
               w1t_ref, w2t_ref, idx_ref, y1_ref, y2_ref):
    xrow = xrow_ref[...]
    dot_hi = functools.partial(
        jax.lax.dot_general,
        dimension_numbers=(((1,), (0,)), ((), ())),
        precision=jax.lax.Precision.HIGHEST,
        preferred_element_type=jnp.float32,
    )
    dot_lo = functools.partial(
        jax.lax.dot_general,
        dimension_numbers=(((1,), (0,)), ((), ())),
        preferred_element_type=jnp.float32,
    )
    y1_ref[...] = dot_hi(xrow, w1t_ref[...])
    y2_ref[...] = dot_hi(xrow, w2t_ref[...])

    rb = xrow.shape[0]
    np_ = xtt_ref.shape[1]
    half = np_ // 2
    xxr = xxr_ref[...]
    inner = dot_lo(xrow, xtt_ref[...])
    xxc = xxc_ref[...]
    colv = jax.lax.broadcasted_iota(jnp.int32, (1, np_), 1)
    d = (xxr + xxc) - 2.0 * inner
    d = jnp.where(colv >= n_valid, _PADV, d)

    a, b = d[:, :half], d[:, half:]
    ia = jax.lax.broadcasted_iota(jnp.int32, (rb, half), 1)
    ib = ia + half
    sel_a = a <= b
    pmin = jnp.where(sel_a, a, b)
    pmax = jnp.where(sel_a, b, a)
    pmin_i = jnp.where(sel_a, ia, ib)
    isum = ia + ib

    new_i = []
    for _ in range(k_sel):
        m = jnp.min(pmin, axis=1, keepdims=True)
        cand = jnp.where(pmin == m, pmin_i, _IBIG)
        am = jnp.min(cand, axis=1, keepdims=True)
        hit = cand == am
        pmin = jnp.where(hit, pmax, pmin)
        pmin_i = jnp.where(hit, isum - pmin_i, pmin_i)
        pmax = jnp.where(hit, _MASKED, pmax)
        new_i.append(am)
    idx_ref[...] = jnp.concatenate(new_i, axis=1)


def _build_topk(np_, c, o, rb, cc, k_sel, n_valid, interpret=False):
    body = functools.partial(_topk_body, n_valid, k_sel, cc)
    return pl.pallas_call(
        body,
        grid=(np_ // rb,),
        in_specs=[
            pl.BlockSpec((rb, c), lambda i: (i, 0)),
            pl.BlockSpec((c, np_), lambda i: (0, 0)),
            pl.BlockSpec((rb, 1), lambda i: (i, 0)),
            pl.BlockSpec((1, np_), lambda i: (0, 0)),
            pl.BlockSpec((c, o), lambda i: (0, 0)),
            pl.BlockSpec((c, o), lambda i: (0, 0)),
        ],
        out_specs=[
            pl.BlockSpec((rb, k_sel), lambda i: (i, 0)),
            pl.BlockSpec((rb, o), lambda i: (i, 0)),
            pl.BlockSpec((rb, o), lambda i: (i, 0)),
        ],
        out_shape=[
            jax.ShapeDtypeStruct((np_, k_sel), jnp.int32),
            jax.ShapeDtypeStruct((np_, o), jnp.float32),
            jax.ShapeDtypeStruct((np_, o), jnp.float32),
        ],
        interpret=interpret,
    )


def _build_sc_gather(np_, o, k_sel, n_valid):
    nw = 32
    rpc = 8
    gpr = o // 16
    rpw = np_ // nw
    nchunks = rpw // rpc
    mesh = plsc.VectorSubcoreMesh(core_axis_name="c", subcore_axis_name="s")

    @functools.partial(
        pl.kernel,
        mesh=mesh,
        out_type=[
            jax.ShapeDtypeStruct((np_, o), jnp.float32),
            jax.ShapeDtypeStruct((np_, o), jnp.float32),
            jax.ShapeDtypeStruct((2 * nw, o), jnp.float32),
        ],
        scratch_types=[
            pltpu.VMEM((rpc * k_sel,), jnp.int32),
            pltpu.VMEM((rpc * k_sel, o), jnp.float32),
            pltpu.VMEM((rpc, o), jnp.float32),
            pltpu.VMEM((rpc, o), jnp.float32),
            pltpu.VMEM((rpc, o), jnp.float32),
            pltpu.VMEM((o,), jnp.float32),
            pltpu.VMEM((o,), jnp.float32),
            pltpu.SemaphoreType.DMA,
        ],
    )
    def sc_gather(y1_hbm, y2_hbm, idx_hbm, mx_hbm, mn_hbm, part_hbm,
                  idx_v, rows_v, y2_v, mx_v, mn_v, s_v, s2_v, sem):
        wid = lax.axis_index("s") * 2 + lax.axis_index("c")
        row0 = wid * rpw
        for g in range(gpr):
            s_v[pl.ds(g * 16, 16)] = jnp.zeros((16,), jnp.float32)
            s2_v[pl.ds(g * 16, 16)] = jnp.zeros((16,), jnp.float32)

        def chunk_body(ch, _):
            n0 = row0 + ch * rpc
            pltpu.sync_copy(idx_hbm.at[pl.ds(n0 * k_sel, rpc * k_sel)],
                            idx_v)
            pltpu.async_copy(y1_hbm.at[idx_v], rows_v, sem).wait()
            pltpu.sync_copy(y2_hbm.at[pl.ds(n0, rpc)], y2_v)

            def row_body(r, valid):
                for g in range(gpr):
                    cs = pl.ds(g * 16, 16)
                    y2g = y2_v[r, cs]
                    z = rows_v[r * k_sel, cs] + y2g
                    mx, mn, s, s2 = z, z, z, z * z
                    for k in range(1, k_sel):
                        z = rows_v[r * k_sel + k, cs] + y2g
                        mx = jnp.maximum(mx, z)
                        mn = jnp.minimum(mn, z)
                        s = s + z
                        s2 = s2 + z * z
                    mx_v[r, cs] = mx
                    mn_v[r, cs] = mn

                    @pl.when(valid)
                    def _():
                        s_v[cs] = s_v[cs] + s
                        s2_v[cs] = s2_v[cs] + s2
                return valid

            lax.fori_loop(0, rpc, row_body, n0 < n_valid)
            pltpu.sync_copy(mx_v, mx_hbm.at[pl.ds(n0, rpc)])
            pltpu.sync_copy(mn_v, mn_hbm.at[pl.ds(n0, rpc)])
            return ()

        lax.fori_loop(0, nchunks, chunk_body, ())
        pltpu.sync_copy(s_v, part_hbm.at[wid])
        pltpu.sync_copy(s2_v, part_hbm.at[nw + wid])

    return sc_gather


def _fin_body(m_count, mx_ref, mn_ref, part_ref, gamma_ref, beta_ref, out_ref):
    part = part_ref[...]
    wk = part.shape[0] // 2
    s = jnp.sum(part[:wk, :], axis=0, keepdims=True)
    s2 = jnp.sum(part[wk:, :], axis=0, keepdims=True)
    mean = s / m_count
    var = s2 / m_count - mean * mean
    scale = gamma_ref[...] * jax.lax.rsqrt(var + _EPS)
    shift = beta_ref[...] - mean * scale
    pick = jnp.where(scale >= 0.0, mx_ref[...], mn_ref[...])
    o = jnp.maximum(pick * scale + shift, 0.0)
    out_ref[0, ...] = o.T


def _build_fin(np_, o, rb, wk, m_count, interpret=False):
    body = functools.partial(_fin_body, float(m_count))
    return pl.pallas_call(
        body,
        grid=(np_ // rb,),
        in_specs=[
            pl.BlockSpec((rb, o), lambda i: (i, 0)),
            pl.BlockSpec((rb, o), lambda i: (i, 0)),
            pl.BlockSpec((2 * wk, o), lambda i: (0, 0)),
            pl.BlockSpec((1, o), lambda i: (0, 0)),
            pl.BlockSpec((1, o), lambda i: (0, 0)),
        ],
        out_specs=pl.BlockSpec((1, o, rb), lambda i: (0, 0, i)),
        out_shape=jax.ShapeDtypeStruct((1, o, np_), jnp.float32),
        interpret=interpret,
    )


def _edge_conv(x, W, gamma, beta, rb, cc, rb_fin, k_sel, interpret=False,
               use_sc=True):
    b, c, n = x.shape
    o = W.shape[0]
    xt = jnp.transpose(x[0])
    np_ = -(-n // cc) * cc
    xtp = jnp.pad(xt, ((0, np_ - n), (0, 0)))
    xtt = jnp.transpose(xtp)
    xx = jnp.sum(xt * xt, axis=1)
    xxr = jnp.pad(xx, (0, np_ - n))[:, None]
    xxc = jnp.pad(xx, (0, np_ - n))[None, :]
    w1t = jnp.transpose(W[:, :c])
    w2t = jnp.transpose(W[:, c:] - W[:, :c])

    idx, y1, y2 = _build_topk(np_, c, o, rb, cc, k_sel, n,
                              interpret=interpret)(xtp, xtt, xxr, xxc,
                                                   w1t, w2t)

    if use_sc:
        idxf = idx.reshape(np_ * k_sel)
        mx, mn, partials = _build_sc_gather(np_, o, k_sel, n)(y1, y2, idxf)
        wk = 32
    else:
        idxv = idx[:n]
        z = y1[idxv] + y2[:n, None, :]
        s = jnp.sum(z, axis=(0, 1))
        s2 = jnp.sum(z * z, axis=(0, 1))
        partials = jnp.stack([s, s2], axis=0)
        mx = jnp.pad(jnp.max(z, axis=1), ((0, np_ - n), (0, 0)))
        mn = jnp.pad(jnp.min(z, axis=1), ((0, np_ - n), (0, 0)))
        wk = 1

    out = _build_fin(np_, o, rb_fin, wk, n * k_sel, interpret=interpret)(
        mx, mn, partials, gamma[None, :], beta[None, :])
    return out[:, :, :n]


def kernel(x, W, gamma, beta):
    return _edge_conv(x, W, gamma, beta, rb=64, cc=2048, rb_fin=512,
                      k_sel=16)


# speedup vs baseline: 1.1567x; 1.0632x over previous
import functools

import jax
import jax.numpy as jnp
from jax import lax
from jax.experimental import pallas as pl
from jax.experimental.pallas import tpu as pltpu
from jax.experimental.pallas import tpu_sc as plsc

_BIG = 2e30
_MASKED = 3e30
_PADV = 1e30
_IBIG = 2**31 - 1
_EPS = 1e-5


def _topk_body(n_valid, k_sel, cc, xrow_ref, xtt_ref, xxr_ref, xxc_ref,
               w1t_ref, w2t_ref, idx_ref, y1_ref, y2_ref):
    xrow = xrow_ref[...]
    dot_hi = functools.partial(
        jax.lax.dot_general,
        dimension_numbers=(((1,), (0,)), ((), ())),
        precision=jax.lax.Precision.HIGHEST,
        preferred_element_type=jnp.float32,
    )
    dot_lo = functools.partial(
        jax.lax.dot_general,
        dimension_numbers=(((1,), (0,)), ((), ())),
        preferred_element_type=jnp.float32,
    )
    y1_ref[...] = dot_hi(xrow, w1t_ref[...])
    y2_ref[...] = dot_hi(xrow, w2t_ref[...])

    rb = xrow.shape[0]
    np_ = xtt_ref.shape[1]
    half = np_ // 2
    xxr = xxr_ref[...]
    inner = dot_lo(xrow, xtt_ref[...])
    xxc = xxc_ref[...]
    colv = jax.lax.broadcasted_iota(jnp.int32, (1, np_), 1)
    d = (xxr + xxc) - 2.0 * inner
    d = jnp.where(colv >= n_valid, _PADV, d)

    ngrp = 2
    grb = rb // ngrp
    ia = jax.lax.broadcasted_iota(jnp.int32, (grb, half), 1)
    ib = ia + half
    isum = ia + ib
    pmin, pmax, pmin_i, new_i = [], [], [], []
    for g in range(ngrp):
        dg = d[g * grb:(g + 1) * grb, :]
        a, b = dg[:, :half], dg[:, half:]
        sel_a = a <= b
        pmin.append(jnp.where(sel_a, a, b))
        pmax.append(jnp.where(sel_a, b, a))
        pmin_i.append(jnp.where(sel_a, ia, ib))
        new_i.append([])
    for _ in range(k_sel):
        for g in range(ngrp):
            m = jnp.min(pmin[g], axis=1, keepdims=True)
            cand = jnp.where(pmin[g] == m, pmin_i[g], _IBIG)
            am = jnp.min(cand, axis=1, keepdims=True)
            hit = cand == am
            pmin[g] = jnp.where(hit, pmax[g], pmin[g])
            pmin_i[g] = jnp.where(hit, isum - pmin_i[g], pmin_i[g])
            pmax[g] = jnp.where(hit, _MASKED, pmax[g])
            new_i[g].append(am)
    for g in range(ngrp):
        idx_ref[g * grb:(g + 1) * grb, :] = jnp.concatenate(new_i[g], axis=1)


def _build_topk(np_, c, o, rb, cc, k_sel, n_valid, interpret=False):
    body = functools.partial(_topk_body, n_valid, k_sel, cc)
    return pl.pallas_call(
        body,
        grid=(np_ // rb,),
        in_specs=[
            pl.BlockSpec((rb, c), lambda i: (i, 0)),
            pl.BlockSpec((c, np_), lambda i: (0, 0)),
            pl.BlockSpec((rb, 1), lambda i: (i, 0)),
            pl.BlockSpec((1, np_), lambda i: (0, 0)),
            pl.BlockSpec((c, o), lambda i: (0, 0)),
            pl.BlockSpec((c, o), lambda i: (0, 0)),
        ],
        out_specs=[
            pl.BlockSpec((rb, k_sel), lambda i: (i, 0)),
            pl.BlockSpec((rb, o), lambda i: (i, 0)),
            pl.BlockSpec((rb, o), lambda i: (i, 0)),
        ],
        out_shape=[
            jax.ShapeDtypeStruct((np_, k_sel), jnp.int32),
            jax.ShapeDtypeStruct((np_, o), jnp.float32),
            jax.ShapeDtypeStruct((np_, o), jnp.float32),
        ],
        interpret=interpret,
    )


def _build_sc_gather(np_, o, k_sel, n_valid):
    nw = 32
    rpc = 8
    gpr = o // 16
    rpw = np_ // nw
    nchunks = rpw // rpc
    mesh = plsc.VectorSubcoreMesh(core_axis_name="c", subcore_axis_name="s")

    @functools.partial(
        pl.kernel,
        mesh=mesh,
        out_type=[
            jax.ShapeDtypeStruct((np_, o), jnp.float32),
            jax.ShapeDtypeStruct((np_, o), jnp.float32),
            jax.ShapeDtypeStruct((2 * nw, o), jnp.float32),
        ],
        scratch_types=[
            pltpu.VMEM((rpc * k_sel,), jnp.int32),
            pltpu.VMEM((rpc * k_sel, o), jnp.float32),
            pltpu.VMEM((rpc, o), jnp.float32),
            pltpu.VMEM((rpc, o), jnp.float32),
            pltpu.VMEM((rpc, o), jnp.float32),
            pltpu.VMEM((o,), jnp.float32),
            pltpu.VMEM((o,), jnp.float32),
            pltpu.SemaphoreType.DMA,
        ],
    )
    def sc_gather(y1_hbm, y2_hbm, idx_hbm, mx_hbm, mn_hbm, part_hbm,
                  idx_v, rows_v, y2_v, mx_v, mn_v, s_v, s2_v, sem):
        wid = lax.axis_index("s") * 2 + lax.axis_index("c")
        row0 = wid * rpw
        for g in range(gpr):
            s_v[pl.ds(g * 16, 16)] = jnp.zeros((16,), jnp.float32)
            s2_v[pl.ds(g * 16, 16)] = jnp.zeros((16,), jnp.float32)

        def chunk_body(ch, _):
            n0 = row0 + ch * rpc
            pltpu.sync_copy(idx_hbm.at[pl.ds(n0 * k_sel, rpc * k_sel)],
                            idx_v)
            pltpu.async_copy(y1_hbm.at[idx_v], rows_v, sem).wait()
            pltpu.sync_copy(y2_hbm.at[pl.ds(n0, rpc)], y2_v)

            def row_body(r, valid):
                for g in range(gpr):
                    cs = pl.ds(g * 16, 16)
                    y2g = y2_v[r, cs]
                    z = rows_v[r * k_sel, cs] + y2g
                    mx, mn, s, s2 = z, z, z, z * z
                    for k in range(1, k_sel):
                        z = rows_v[r * k_sel + k, cs] + y2g
                        mx = jnp.maximum(mx, z)
                        mn = jnp.minimum(mn, z)
                        s = s + z
                        s2 = s2 + z * z
                    mx_v[r, cs] = mx
                    mn_v[r, cs] = mn

                    @pl.when(valid)
                    def _():
                        s_v[cs] = s_v[cs] + s
                        s2_v[cs] = s2_v[cs] + s2
                return valid

            lax.fori_loop(0, rpc, row_body, n0 < n_valid)
            pltpu.sync_copy(mx_v, mx_hbm.at[pl.ds(n0, rpc)])
            pltpu.sync_copy(mn_v, mn_hbm.at[pl.ds(n0, rpc)])
            return ()

        lax.fori_loop(0, nchunks, chunk_body, ())
        pltpu.sync_copy(s_v, part_hbm.at[wid])
        pltpu.sync_copy(s2_v, part_hbm.at[nw + wid])

    return sc_gather


def _fin_body(m_count, mx_ref, mn_ref, part_ref, gamma_ref, beta_ref, out_ref):
    part = part_ref[...]
    wk = part.shape[0] // 2
    s = jnp.sum(part[:wk, :], axis=0, keepdims=True)
    s2 = jnp.sum(part[wk:, :], axis=0, keepdims=True)
    mean = s / m_count
    var = s2 / m_count - mean * mean
    scale = gamma_ref[...] * jax.lax.rsqrt(var + _EPS)
    shift = beta_ref[...] - mean * scale
    pick = jnp.where(scale >= 0.0, mx_ref[...], mn_ref[...])
    o = jnp.maximum(pick * scale + shift, 0.0)
    out_ref[0, ...] = o.T


def _build_fin(np_, o, rb, wk, m_count, interpret=False):
    body = functools.partial(_fin_body, float(m_count))
    return pl.pallas_call(
        body,
        grid=(np_ // rb,),
        in_specs=[
            pl.BlockSpec((rb, o), lambda i: (i, 0)),
            pl.BlockSpec((rb, o), lambda i: (i, 0)),
            pl.BlockSpec((2 * wk, o), lambda i: (0, 0)),
            pl.BlockSpec((1, o), lambda i: (0, 0)),
            pl.BlockSpec((1, o), lambda i: (0, 0)),
        ],
        out_specs=pl.BlockSpec((1, o, rb), lambda i: (0, 0, i)),
        out_shape=jax.ShapeDtypeStruct((1, o, np_), jnp.float32),
        interpret=interpret,
    )


def _edge_conv(x, W, gamma, beta, rb, cc, rb_fin, k_sel, interpret=False,
               use_sc=True):
    b, c, n = x.shape
    o = W.shape[0]
    xt = jnp.transpose(x[0])
    np_ = -(-n // cc) * cc
    xtp = jnp.pad(xt, ((0, np_ - n), (0, 0)))
    xtt = jnp.transpose(xtp)
    xx = jnp.sum(xt * xt, axis=1)
    xxr = jnp.pad(xx, (0, np_ - n))[:, None]
    xxc = jnp.pad(xx, (0, np_ - n))[None, :]
    w1t = jnp.transpose(W[:, :c])
    w2t = jnp.transpose(W[:, c:] - W[:, :c])

    idx, y1, y2 = _build_topk(np_, c, o, rb, cc, k_sel, n,
                              interpret=interpret)(xtp, xtt, xxr, xxc,
                                                   w1t, w2t)

    if use_sc:
        idxf = idx.reshape(np_ * k_sel)
        mx, mn, partials = _build_sc_gather(np_, o, k_sel, n)(y1, y2, idxf)
        wk = 32
    else:
        idxv = idx[:n]
        z = y1[idxv] + y2[:n, None, :]
        s = jnp.sum(z, axis=(0, 1))
        s2 = jnp.sum(z * z, axis=(0, 1))
        partials = jnp.stack([s, s2], axis=0)
        mx = jnp.pad(jnp.max(z, axis=1), ((0, np_ - n), (0, 0)))
        mn = jnp.pad(jnp.min(z, axis=1), ((0, np_ - n), (0, 0)))
        wk = 1

    out = _build_fin(np_, o, rb_fin, wk, n * k_sel, interpret=interpret)(
        mx, mn, partials, gamma[None, :], beta[None, :])
    return out[:, :, :n]


def kernel(x, W, gamma, beta):
    return _edge_conv(x, W, gamma, beta, rb=128, cc=2048, rb_fin=512,
                      k_sel=16)
